# baseline (speedup 1.0000x reference)
import jax
import jax.numpy as jnp
from jax import lax
from jax.experimental import pallas as pl
from jax.experimental.pallas import tpu as pltpu
from jax.experimental.pallas import tpu_sc as plsc

_N_EXPERTS = 64
_D = 64
_B = 16384

_NC = 2
_NS = 16
_NW = _NC * _NS

_TI = _D // 8
_NQ = _NW // _TI
_BQ = _B // _NQ // 128
_BPW = _B // _NQ


def _rsqrt_newton(s):
    i = lax.bitcast_convert_type(s, jnp.int32)
    y = lax.bitcast_convert_type(jnp.int32(0x5F3759DF) - (i >> 1), jnp.float32)
    for _ in range(4):
        y = y * (1.5 - 0.5 * s * y * y)
    return y


_mesh = plsc.VectorSubcoreMesh(
    core_axis_name="c", subcore_axis_name="s", num_cores=_NC, num_subcores=_NS
)

_KERNEL_KWARGS = dict(
    mesh=_mesh,
    out_type=jax.ShapeDtypeStruct((_TI * 128, 8, 128), jnp.float32),
    scratch_types=[
        pltpu.VMEM((_N_EXPERTS, _D + 1), jnp.float32),
        pltpu.VMEM((_BPW,), jnp.int32),
        pltpu.VMEM((_BQ, 8, 128), jnp.float32),
        pltpu.SemaphoreType.DMA,
        pltpu.SemaphoreType.DMA,
        pltpu.SemaphoreType.DMA,
        pltpu.SemaphoreType.DMA,
    ],
    compiler_params=pltpu.CompilerParams(
        use_tc_tiling_on_sc=False, needs_layout_passes=False
    ),
)


def _lookup_body(tab_hbm, idx_hbm, out_hbm, tab_v, idx_v, stage_v,
                 sem_t, sem_i, sem_a, sem_b):
    wid = lax.axis_index("s") * _NC + lax.axis_index("c")
    ti = wid // _NQ
    q = wid % _NQ
    idx_dma = pltpu.async_copy(idx_hbm.at[pl.ds(q * _BPW, _BPW)], idx_v, sem_i)
    pltpu.async_copy(tab_hbm, tab_v.at[:, pl.ds(0, _D)], sem_t).wait()

    lanes = lax.iota(jnp.int32, 16)

    @plsc.parallel_loop(0, _N_EXPERTS // 16, unroll=4)
    def normalize(g):
        rows16 = lanes + g * 16
        acc = jnp.zeros((16,), jnp.float32)
        for d in range(_D):
            dfull = jnp.full((16,), d, jnp.int32)
            v = plsc.load_gather(tab_v, [rows16, dfull])
            acc = acc + v * v
        scale = jnp.where(acc > 0.0, _rsqrt_newton(acc), 0.0)
        for d in range(_D):
            dfull = jnp.full((16,), d, jnp.int32)
            v = plsc.load_gather(tab_v, [rows16, dfull])
            plsc.store_scatter(tab_v, [rows16, dfull], v * scale)

    idx_dma.wait()

    def build_tile(t):
        for j in range(8):
            idx16 = idx_v[pl.ds(t * 128 + j * 16, 16)]
            for s in range(8):
                dfull = jnp.full((16,), ti * 8 + s, jnp.int32)
                stage_v[t, s, pl.ds(j * 16, 16)] = plsc.load_gather(
                    tab_v, [idx16, dfull]
                )

    base = ti * 128 + q * _BQ
    half = _BQ // 2
    plsc.parallel_loop(0, half, unroll=8)(build_tile)
    dma_a = pltpu.async_copy(
        stage_v.at[pl.ds(0, half)], out_hbm.at[pl.ds(base, half)], sem_a
    )
    plsc.parallel_loop(half, _BQ, unroll=8)(build_tile)
    dma_b = pltpu.async_copy(
        stage_v.at[pl.ds(half, half)], out_hbm.at[pl.ds(base + half, half)],
        sem_b,
    )
    dma_a.wait()
    dma_b.wait()


_lookup_kernel = pl.kernel(_lookup_body, **_KERNEL_KWARGS)


def kernel(experts, table):
    out4 = _lookup_kernel(table, experts.astype(jnp.int32))
    out4 = out4.reshape(_TI, 128, 8, 128)
    return out4.transpose(1, 3, 0, 2).reshape(_B, _D)


# speedup vs baseline: 1.4143x; 1.4143x over previous
import jax
import jax.numpy as jnp
from jax import lax
from jax.experimental import pallas as pl
from jax.experimental.pallas import tpu as pltpu
from jax.experimental.pallas import tpu_sc as plsc

_N_EXPERTS = 64
_D = 64
_B = 16384

_NC = 2
_NS = 16
_NW = _NC * _NS

_TI = _D // 8
_NQ = _NW // _TI
_BQ = _B // _NQ // 128
_BPW = _B // _NQ


def _rsqrt_newton(s):
    i = lax.bitcast_convert_type(s, jnp.int32)
    y = lax.bitcast_convert_type(jnp.int32(0x5F3759DF) - (i >> 1), jnp.float32)
    for _ in range(4):
        y = y * (1.5 - 0.5 * s * y * y)
    return y


_mesh = plsc.VectorSubcoreMesh(
    core_axis_name="c", subcore_axis_name="s", num_cores=_NC, num_subcores=_NS
)

_KERNEL_KWARGS = dict(
    mesh=_mesh,
    out_type=jax.ShapeDtypeStruct((_TI * 128, 8, 128), jnp.float32),
    scratch_types=[
        pltpu.VMEM((_N_EXPERTS, _D + 1), jnp.float32),
        pltpu.VMEM((_BPW,), jnp.int32),
        pltpu.VMEM((_BQ, 8, 128), jnp.float32),
        pltpu.SemaphoreType.DMA,
        pltpu.SemaphoreType.DMA,
        pltpu.SemaphoreType.DMA,
        pltpu.SemaphoreType.DMA,
    ],
    compiler_params=pltpu.CompilerParams(
        use_tc_tiling_on_sc=False, needs_layout_passes=False
    ),
)


def _lookup_body(tab_hbm, idx_hbm, out_hbm, tab_v, idx_v, stage_v,
                 sem_t, sem_i, sem_a, sem_b):
    wid = lax.axis_index("s") * _NC + lax.axis_index("c")
    ti = wid // _NQ
    q = wid % _NQ
    idx_dma = pltpu.async_copy(idx_hbm.at[pl.ds(q * _BPW, _BPW)], idx_v, sem_i)
    pltpu.async_copy(tab_hbm, tab_v.at[:, pl.ds(0, _D)], sem_t).wait()

    @plsc.parallel_loop(0, _N_EXPERTS, unroll=2)
    def normalize(r):
        vs = [tab_v[r, pl.ds(d * 16, 16)] for d in range(_D // 16)]
        sq = vs[0] * vs[0]
        for v in vs[1:]:
            sq = sq + v * v
        s = lax.reduce_sum_p.bind(sq, axes=(0,))
        scale = jnp.where(s > 0.0, _rsqrt_newton(s), 0.0)
        for d, v in enumerate(vs):
            tab_v[r, pl.ds(d * 16, 16)] = v * scale

    idx_dma.wait()

    def build_group(tj):
        t = tj // 8
        j = tj % 8
        idx16 = idx_v[pl.ds(tj * 16, 16)]
        for s in range(8):
            dfull = jnp.full((16,), ti * 8 + s, jnp.int32)
            stage_v[t, s, pl.ds(j * 16, 16)] = plsc.load_gather(
                tab_v, [idx16, dfull]
            )

    base = ti * 128 + q * _BQ
    half = _BQ // 2
    plsc.parallel_loop(0, half * 8, unroll=4)(build_group)
    dma_a = pltpu.async_copy(
        stage_v.at[pl.ds(0, half)], out_hbm.at[pl.ds(base, half)], sem_a
    )
    plsc.parallel_loop(half * 8, _BQ * 8, unroll=4)(build_group)
    dma_b = pltpu.async_copy(
        stage_v.at[pl.ds(half, half)], out_hbm.at[pl.ds(base + half, half)],
        sem_b,
    )
    dma_a.wait()
    dma_b.wait()


_lookup_kernel = pl.kernel(_lookup_body, **_KERNEL_KWARGS)


def kernel(experts, table):
    out4 = _lookup_kernel(table, experts.astype(jnp.int32))
    out4 = out4.reshape(_TI, 128, 8, 128)
    return out4.transpose(1, 3, 0, 2).reshape(_B, _D)
